# baseline (speedup 1.0000x reference)
import jax
import jax.numpy as jnp
from jax.experimental import pallas as pl

N = 2048
D = 64
K = 3
_BIG = 3.4e38


def _ln(x, w, b):
    m = x.mean(-1, keepdims=True)
    v = ((x - m) ** 2).mean(-1, keepdims=True)
    return (x - m) * jax.lax.rsqrt(v + 1e-5) * w + b


def _fused_body(feat_ref, pos_ref, post_ref, n1w_ref, n1b_ref, n2w_ref,
                n2b_ref, w1_ref, b1_ref, lnw_ref, lnb_ref, w2_ref, b2_ref,
                out_ref):
    f32 = jnp.float32
    bf16 = jnp.bfloat16
    feat = feat_ref[...]
    x1 = _ln(feat, n1w_ref[...], n1b_ref[...])
    x2 = _ln(x1 + x1, n2w_ref[...], n2b_ref[...])

    pos = pos_ref[...]
    post = post_ref[...]
    dot = jax.lax.dot_general(pos, post, (((1,), (0,)), ((), ())),
                              preferred_element_type=f32)
    sq_r = jnp.sum(pos * pos, axis=1, keepdims=True)
    sq_c = jnp.sum(post * post, axis=0, keepdims=True)
    d2 = sq_r + sq_c - 2.0 * dot
    rvec = jax.lax.broadcasted_iota(jnp.int32, (N, 1), 0)
    cols = jax.lax.broadcasted_iota(jnp.int32, (N, N), 1)
    d2 = jnp.where(cols == rvec, _BIG, d2)

    idxs = []
    excl = None
    for _ in range(K):
        masked = d2 if excl is None else jnp.where(excl, _BIG, d2)
        m = jnp.min(masked, axis=1, keepdims=True)
        hit = d2 == m if excl is None else (d2 == m) & ~excl
        am = jnp.min(jnp.where(hit, cols, N), axis=1, keepdims=True)
        idxs.append(am)
        sel = cols == am
        excl = sel if excl is None else excl | sel

    inv_norm = jax.lax.rsqrt(jnp.maximum(
        jnp.sum(x2 * x2, axis=1, keepdims=True), 1e-16))
    xn = x2 * inv_norm
    csim = jax.lax.dot_general(xn, xn, (((1,), (1,)), ((), ())),
                               preferred_element_type=f32)
    sig = jax.nn.sigmoid(csim)

    adj_f = (jnp.where(cols == idxs[0], sig, 0.0)
             + jnp.where(cols == idxs[1], sig, 0.0)
             + jnp.where(cols == idxs[2], sig, 0.0))
    deg = jnp.sum(adj_f, axis=1, keepdims=True)
    adj = adj_f.astype(bf16)

    dis1 = jax.lax.rsqrt(deg + 3.0)
    h1 = jax.lax.dot_general(x2, w1_ref[...], (((1,), (0,)), ((), ())),
                             preferred_element_type=f32)
    agg1 = jax.lax.dot_general(adj, (dis1 * h1).astype(bf16),
                               (((1,), (0,)), ((), ())),
                               preferred_element_type=f32)
    out1 = dis1 * agg1 + 3.0 * dis1 * dis1 * h1 + b1_ref[...]
    y = jax.nn.relu(_ln(out1, lnw_ref[...], lnb_ref[...]))

    dis2 = jax.lax.rsqrt(deg + 2.0)
    h2 = jax.lax.dot_general(y, w2_ref[...], (((1,), (0,)), ((), ())),
                             preferred_element_type=f32)
    agg2 = jax.lax.dot_general(adj, (dis2 * h2).astype(bf16),
                               (((1,), (0,)), ((), ())),
                               preferred_element_type=f32)
    out2 = dis2 * agg2 + 2.0 * dis2 * dis2 * h2 + b2_ref[...]

    out_ref[...] = x2 + x2 + out2


def kernel(sp_center_feat, edge_index_tran, edge_attr_rpe, norm_index,
           sp_crood, norm1_w, norm1_b, norm2_w, norm2_b, W1, b1, ln_w, ln_b,
           W2, b2):
    del edge_index_tran, edge_attr_rpe, norm_index
    pos = jnp.zeros((N, 8), jnp.float32).at[:, :3].set(sp_crood)
    post = pos.T
    row = lambda v: v.reshape(1, D)
    return pl.pallas_call(
        _fused_body,
        out_shape=jax.ShapeDtypeStruct((N, D), jnp.float32),
    )(sp_center_feat, pos, post, row(norm1_w), row(norm1_b), row(norm2_w),
      row(norm2_b), W1, row(b1), row(ln_w), row(ln_b), W2, row(b2))


# speedup vs baseline: 1.1098x; 1.1098x over previous
import jax
import jax.numpy as jnp
from jax.experimental import pallas as pl

N = 2048
D = 64
K = 3
_BIG = 3.4e38


def _ln(x, w, b):
    m = x.mean(-1, keepdims=True)
    v = ((x - m) ** 2).mean(-1, keepdims=True)
    return (x - m) * jax.lax.rsqrt(v + 1e-5) * w + b


def _fused_body(feat_ref, pos_ref, post_ref, n1w_ref, n1b_ref, n2w_ref,
                n2b_ref, w1_ref, b1_ref, lnw_ref, lnb_ref, w2_ref, b2_ref,
                out_ref):
    f32 = jnp.float32
    bf16 = jnp.bfloat16
    feat = feat_ref[...]
    x1 = _ln(feat, n1w_ref[...], n1b_ref[...])
    x2 = _ln(x1 + x1, n2w_ref[...], n2b_ref[...])

    pos = pos_ref[...]
    post = post_ref[...]
    dot = jax.lax.dot_general(pos, post, (((1,), (0,)), ((), ())),
                              preferred_element_type=f32)
    sq_r = jnp.sum(pos * pos, axis=1, keepdims=True)
    sq_c = jnp.sum(post * post, axis=0, keepdims=True)
    d2 = sq_r + sq_c - 2.0 * dot
    rows = jax.lax.broadcasted_iota(jnp.int32, (N, N), 0)
    cols = jax.lax.broadcasted_iota(jnp.int32, (N, N), 1)
    d2 = jnp.where(rows == cols, _BIG, d2)

    idxs = []
    for k in range(K):
        m = jnp.min(d2, axis=1, keepdims=True)
        am = jnp.min(jnp.where(d2 == m, cols, N), axis=1, keepdims=True)
        idxs.append(am)
        if k + 1 < K:
            d2 = jnp.where(cols == am, _BIG, d2)

    inv_norm = jax.lax.rsqrt(jnp.maximum(
        jnp.sum(x2 * x2, axis=1, keepdims=True), 1e-16))
    xn = x2 * inv_norm
    xnb = xn.astype(bf16)
    csim = jax.lax.dot_general(xnb, xnb, (((1,), (1,)), ((), ())),
                               preferred_element_type=f32)
    sims = [jax.nn.sigmoid(jnp.sum(
        jnp.where(cols == am, csim, 0.0), axis=1, keepdims=True))
        for am in idxs]
    deg = sims[0] + sims[1] + sims[2]

    adj = (jnp.where(cols == idxs[0], sims[0], 0.0)
           + jnp.where(cols == idxs[1], sims[1], 0.0)
           + jnp.where(cols == idxs[2], sims[2], 0.0)).astype(bf16)

    dis1 = jax.lax.rsqrt(deg + 3.0)
    h1 = jax.lax.dot_general(x2, w1_ref[...], (((1,), (0,)), ((), ())),
                             preferred_element_type=f32)
    agg1 = jax.lax.dot_general(adj, (dis1 * h1).astype(bf16),
                               (((1,), (0,)), ((), ())),
                               preferred_element_type=f32)
    out1 = dis1 * agg1 + 3.0 * dis1 * dis1 * h1 + b1_ref[...]
    y = jax.nn.relu(_ln(out1, lnw_ref[...], lnb_ref[...]))

    dis2 = jax.lax.rsqrt(deg + 2.0)
    h2 = jax.lax.dot_general(y, w2_ref[...], (((1,), (0,)), ((), ())),
                             preferred_element_type=f32)
    agg2 = jax.lax.dot_general(adj, (dis2 * h2).astype(bf16),
                               (((1,), (0,)), ((), ())),
                               preferred_element_type=f32)
    out2 = dis2 * agg2 + 2.0 * dis2 * dis2 * h2 + b2_ref[...]

    out_ref[...] = x2 + x2 + out2


def kernel(sp_center_feat, edge_index_tran, edge_attr_rpe, norm_index,
           sp_crood, norm1_w, norm1_b, norm2_w, norm2_b, W1, b1, ln_w, ln_b,
           W2, b2):
    del edge_index_tran, edge_attr_rpe, norm_index
    pos = jnp.zeros((N, 8), jnp.float32).at[:, :3].set(sp_crood)
    post = pos.T
    row = lambda v: v.reshape(1, D)
    return pl.pallas_call(
        _fused_body,
        out_shape=jax.ShapeDtypeStruct((N, D), jnp.float32),
    )(sp_center_feat, pos, post, row(norm1_w), row(norm1_b), row(norm2_w),
      row(norm2_b), W1, row(b1), row(ln_w), row(ln_b), W2, row(b2))


# speedup vs baseline: 1.1559x; 1.0416x over previous
import jax
import jax.numpy as jnp
from jax.experimental import pallas as pl

N = 2048
D = 64
K = 3
_BIG = 3.4e38


def _ln(x, w, b):
    m = x.mean(-1, keepdims=True)
    v = ((x - m) ** 2).mean(-1, keepdims=True)
    return (x - m) * jax.lax.rsqrt(v + 1e-5) * w + b


def _fused_body(feat_ref, pos_ref, post_ref, n1w_ref, n1b_ref, n2w_ref,
                n2b_ref, w1_ref, b1_ref, lnw_ref, lnb_ref, w2_ref, b2_ref,
                out_ref):
    f32 = jnp.float32
    bf16 = jnp.bfloat16
    feat = feat_ref[...]
    x1 = _ln(feat, n1w_ref[...], n1b_ref[...])
    x2 = _ln(x1 + x1, n2w_ref[...], n2b_ref[...])

    pos = pos_ref[...]
    post = post_ref[...]
    dot = jax.lax.dot_general(pos, post, (((1,), (0,)), ((), ())),
                              preferred_element_type=f32)
    sq_r = jnp.sum(pos * pos, axis=1, keepdims=True)
    sq_c = jnp.sum(post * post, axis=0, keepdims=True)
    d2 = sq_r + sq_c - 2.0 * dot
    rows = jax.lax.broadcasted_iota(jnp.int32, (N, 1), 0).astype(f32)
    cols = jax.lax.broadcasted_iota(jnp.int32, (N, N), 1).astype(f32)
    d2 = jnp.where(cols == rows, _BIG, d2)

    idxs = []
    for k in range(K):
        m = jnp.min(d2, axis=1, keepdims=True)
        am = jnp.min(jnp.where(d2 == m, cols, float(N)), axis=1,
                     keepdims=True)
        idxs.append(am)
        if k + 1 < K:
            d2 = jnp.where(cols == am, _BIG, d2)

    inv_norm = jax.lax.rsqrt(jnp.maximum(
        jnp.sum(x2 * x2, axis=1, keepdims=True), 1e-16))
    xn = x2 * inv_norm
    xnb = xn.astype(bf16)
    csim = jax.lax.dot_general(xnb, xnb, (((1,), (1,)), ((), ())),
                               preferred_element_type=f32)
    sims = [jax.nn.sigmoid(jnp.sum(
        jnp.where(cols == am, csim, 0.0), axis=1, keepdims=True))
        for am in idxs]
    deg = sims[0] + sims[1] + sims[2]

    adj = (jnp.where(cols == idxs[0], sims[0], 0.0)
           + jnp.where(cols == idxs[1], sims[1], 0.0)
           + jnp.where(cols == idxs[2], sims[2], 0.0)).astype(bf16)

    dis1 = jax.lax.rsqrt(deg + 3.0)
    h1 = jax.lax.dot_general(x2, w1_ref[...], (((1,), (0,)), ((), ())),
                             preferred_element_type=f32)
    agg1 = jax.lax.dot_general(adj, (dis1 * h1).astype(bf16),
                               (((1,), (0,)), ((), ())),
                               preferred_element_type=f32)
    out1 = dis1 * agg1 + 3.0 * dis1 * dis1 * h1 + b1_ref[...]
    y = jax.nn.relu(_ln(out1, lnw_ref[...], lnb_ref[...]))

    dis2 = jax.lax.rsqrt(deg + 2.0)
    h2 = jax.lax.dot_general(y, w2_ref[...], (((1,), (0,)), ((), ())),
                             preferred_element_type=f32)
    agg2 = jax.lax.dot_general(adj, (dis2 * h2).astype(bf16),
                               (((1,), (0,)), ((), ())),
                               preferred_element_type=f32)
    out2 = dis2 * agg2 + 2.0 * dis2 * dis2 * h2 + b2_ref[...]

    out_ref[...] = x2 + x2 + out2


def kernel(sp_center_feat, edge_index_tran, edge_attr_rpe, norm_index,
           sp_crood, norm1_w, norm1_b, norm2_w, norm2_b, W1, b1, ln_w, ln_b,
           W2, b2):
    del edge_index_tran, edge_attr_rpe, norm_index
    pos = jnp.zeros((N, 8), jnp.float32).at[:, :3].set(sp_crood)
    post = pos.T
    row = lambda v: v.reshape(1, D)
    return pl.pallas_call(
        _fused_body,
        out_shape=jax.ShapeDtypeStruct((N, D), jnp.float32),
    )(sp_center_feat, pos, post, row(norm1_w), row(norm1_b), row(norm2_w),
      row(norm2_b), W1, row(b1), row(ln_w), row(ln_b), W2, row(b2))
